# Initial kernel scaffold; baseline (speedup 1.0000x reference)
import jax
import jax.numpy as jnp
from jax.experimental import pallas as pl


def kernel(t_f, f_feats, seg_f, t_r, r_feats, seg_r, t_m, m_feats, seg_m, W_t2v, b_t2v, Wf1, bf1, Wf2, bf2, Wr1, br1, Wr2, br2, Wm1, bm1, Wm2, bm2, W_combo, b_combo):
    raise NotImplementedError("write your pallas kernel here")


# speedup vs baseline: 1.5899x; 1.5899x over previous
import functools

import jax
import jax.numpy as jnp
from jax.experimental import pallas as pl
from jax.experimental.pallas import tpu as pltpu

_SLOPE = (1.0 / 8.0 + 1.0 / 3.0) / 2.0
_B = 16
_BLK = 4096


def _rr(x):
    return jnp.where(x >= 0, x, x * _SLOPE)


def _dot(a, b):
    return jax.lax.dot_general(a, b, (((1,), (0,)), ((), ())),
                               preferred_element_type=jnp.float32)


def _dott(a, b):
    return jax.lax.dot_general(a, b, (((0,), (0,)), ((), ())),
                               preferred_element_type=jnp.float32)


def _aggr_kernel(nb,
                 t_f_ref, xf_ref, sf_ref,
                 t_r_ref, xr_ref, sr_ref,
                 t_m_ref, xm_ref, sm_ref,
                 wtv_ref, btv_ref,
                 w1t_f_ref, w1x_f_ref, b1f_ref, w2f_ref, b2f_ref,
                 w1t_r_ref, w1x_r_ref, b1r_ref, w2r_ref, b2r_ref,
                 w1t_m_ref, w1x_m_ref, b1m_ref, w2m_ref, b2m_ref,
                 wcf_ref, wcr_ref, wcm_ref, bc_ref,
                 out_ref,
                 sums_f, sums_r, sums_m, cnt_f, cnt_r, cnt_m):
    i = pl.program_id(0)

    @pl.when(i == 0)
    def _init():
        sums_f[:, :] = jnp.zeros_like(sums_f)
        sums_r[:, :] = jnp.zeros_like(sums_r)
        sums_m[:, :] = jnp.zeros_like(sums_m)
        cnt_f[:, :] = jnp.zeros_like(cnt_f)
        cnt_r[:, :] = jnp.zeros_like(cnt_r)
        cnt_m[:, :] = jnp.zeros_like(cnt_m)

    wtv = wtv_ref[:, :]
    btv = btv_ref[:, :]

    def one(t_ref, x_ref, s_ref, w1t_ref, w1x_ref, b1_ref, w2_ref, b2_ref,
            sums, cnt):
        t = t_ref[:, :]
        a = t * wtv + btv
        lane = jax.lax.broadcasted_iota(jnp.int32, a.shape, 1)
        t2v = jnp.where(lane == 0, a, jnp.sin(a))
        h = _dot(t2v, w1t_ref[:, :]) + _dot(x_ref[:, :], w1x_ref[:, :])
        h = _rr(h + b1_ref[:, :])
        h = _rr(_dot(h, w2_ref[:, :]) + b2_ref[:, :])
        seg = s_ref[:, :]
        iota = jax.lax.broadcasted_iota(jnp.int32, (seg.shape[0], _B), 1)
        oh = (seg == iota).astype(jnp.float32)
        sums[:, :] += _dott(oh, h)
        ones = jnp.ones((seg.shape[0], 1), jnp.float32)
        cnt[:, :] += _dott(oh, ones)

    one(t_f_ref, xf_ref, sf_ref, w1t_f_ref, w1x_f_ref, b1f_ref, w2f_ref,
        b2f_ref, sums_f, cnt_f)
    one(t_r_ref, xr_ref, sr_ref, w1t_r_ref, w1x_r_ref, b1r_ref, w2r_ref,
        b2r_ref, sums_r, cnt_r)
    one(t_m_ref, xm_ref, sm_ref, w1t_m_ref, w1x_m_ref, b1m_ref, w2m_ref,
        b2m_ref, sums_m, cnt_m)

    @pl.when(i == nb - 1)
    def _fin():
        mf = sums_f[:, :] / jnp.maximum(cnt_f[:, :], 1.0)
        mr = sums_r[:, :] / jnp.maximum(cnt_r[:, :], 1.0)
        mm = sums_m[:, :] / jnp.maximum(cnt_m[:, :], 1.0)
        y = (_dot(mf, wcf_ref[:, :]) + _dot(mr, wcr_ref[:, :])
             + _dot(mm, wcm_ref[:, :]) + bc_ref[:, :])
        out_ref[:, :] = _rr(y)


def kernel(t_f, f_feats, seg_f, t_r, r_feats, seg_r, t_m, m_feats, seg_m,
           W_t2v, b_t2v, Wf1, bf1, Wf2, bf2, Wr1, br1, Wr2, br2,
           Wm1, bm1, Wm2, bm2, W_combo, b_combo):
    n = t_f.shape[0]
    blk = _BLK
    while n % blk:
        blk //= 2
    nb = n // blk
    t2v = W_t2v.shape[0]
    out_dim = W_combo.shape[0]
    hid = Wf2.shape[1]
    o3 = Wf2.shape[0]

    sf = seg_f.astype(jnp.int32).reshape(n, 1)
    sr = seg_r.astype(jnp.int32).reshape(n, 1)
    sm = seg_m.astype(jnp.int32).reshape(n, 1)

    wtv = W_t2v.reshape(1, t2v)
    btv = b_t2v.reshape(1, t2v)

    def prep(W1, b1, W2, b2):
        return (W1[:, :t2v].T, W1[:, t2v:].T, b1.reshape(1, -1),
                W2.T, b2.reshape(1, -1))

    w1t_f, w1x_f, b1f, w2f, b2f = prep(Wf1, bf1, Wf2, bf2)
    w1t_r, w1x_r, b1r, w2r, b2r = prep(Wr1, br1, Wr2, br2)
    w1t_m, w1x_m, b1m, w2m, b2m = prep(Wm1, bm1, Wm2, bm2)

    wcf = W_combo[:, :o3].T
    wcr = W_combo[:, o3:2 * o3].T
    wcm = W_combo[:, 2 * o3:].T
    bc = b_combo.reshape(1, -1)

    tok = lambda i: (i, 0)
    fix = lambda i: (0, 0)

    def tspec():
        return pl.BlockSpec((blk, 1), tok)

    def xspec(w):
        return pl.BlockSpec((blk, w), tok)

    def wspec(a):
        return pl.BlockSpec(a.shape, fix)

    in_specs = [
        tspec(), xspec(f_feats.shape[1]), tspec(),
        tspec(), xspec(r_feats.shape[1]), tspec(),
        tspec(), xspec(m_feats.shape[1]), tspec(),
        wspec(wtv), wspec(btv),
        wspec(w1t_f), wspec(w1x_f), wspec(b1f), wspec(w2f), wspec(b2f),
        wspec(w1t_r), wspec(w1x_r), wspec(b1r), wspec(w2r), wspec(b2r),
        wspec(w1t_m), wspec(w1x_m), wspec(b1m), wspec(w2m), wspec(b2m),
        wspec(wcf), wspec(wcr), wspec(wcm), wspec(bc),
    ]

    out = pl.pallas_call(
        functools.partial(_aggr_kernel, nb),
        grid=(nb,),
        in_specs=in_specs,
        out_specs=pl.BlockSpec((_B, out_dim), fix),
        out_shape=jax.ShapeDtypeStruct((_B, out_dim), jnp.float32),
        scratch_shapes=[
            pltpu.VMEM((_B, o3), jnp.float32),
            pltpu.VMEM((_B, o3), jnp.float32),
            pltpu.VMEM((_B, o3), jnp.float32),
            pltpu.VMEM((_B, 1), jnp.float32),
            pltpu.VMEM((_B, 1), jnp.float32),
            pltpu.VMEM((_B, 1), jnp.float32),
        ],
        compiler_params=pltpu.CompilerParams(
            dimension_semantics=("arbitrary",)),
    )(t_f, f_feats, sf, t_r, r_feats, sr, t_m, m_feats, sm,
      wtv, btv,
      w1t_f, w1x_f, b1f, w2f, b2f,
      w1t_r, w1x_r, b1r, w2r, b2r,
      w1t_m, w1x_m, b1m, w2m, b2m,
      wcf, wcr, wcm, bc)
    return out
